# baseline (speedup 1.0000x reference)
import functools

import jax
import jax.numpy as jnp
from jax.experimental import pallas as pl
from jax.experimental.pallas import tpu as pltpu

N_TRAIN = 20000
W = 256
N_TEST = 1024
BC = 2048
N_PAD = 20480
N_TILES = N_PAD // BC
BT = 512
RB = 128
LANES = 128
CHUNKS = BC // LANES
CLASS_SIZE = 2000
NUM_CLASSES = 10
MAX_DISTANCE = 256
SMALL = -(2 ** 30)


def _body(test_ref, train_ref, colv_ref, out_ref, t1, t2, t3):
    t = pl.program_id(0)
    base = pl.program_id(1) * BT
    dot = jax.lax.dot_general(
        test_ref[...], train_ref[...],
        (((1,), (1,)), ((), ())),
        preferred_element_type=jnp.int32,
    )

    @pl.when(t == 0)
    def _():
        sl = pl.ds(base, BT)
        t1[sl, :] = jnp.full((BT, LANES), SMALL, jnp.int32)
        t2[sl, :] = jnp.full((BT, LANES), SMALL, jnp.int32)
        t3[sl, :] = jnp.full((BT, LANES), SMALL, jnp.int32)

    for rb in range(BT // RB):
        sl = pl.ds(base + rb * RB, RB)
        b1, b2, b3 = t1[sl, :], t2[sl, :], t3[sl, :]
        for c in range(CHUNKS):
            x = ((dot[rb * RB:(rb + 1) * RB, c * LANES:(c + 1) * LANES] << 15)
                 - colv_ref[:, c * LANES:(c + 1) * LANES])
            hi = jnp.maximum(b1, x)
            lo = jnp.minimum(b1, x)
            b1 = hi
            hi = jnp.maximum(b2, lo)
            lo = jnp.minimum(b2, lo)
            b2 = hi
            b3 = jnp.maximum(b3, lo)
        t1[sl, :] = b1
        t2[sl, :] = b2
        t3[sl, :] = b3

    @pl.when(t == N_TILES - 1)
    def _():
        sl = pl.ds(base, BT)
        cand = jnp.concatenate([t1[sl, :], t2[sl, :], t3[sl, :]], axis=1)
        m1 = jnp.max(cand, axis=1, keepdims=True)
        c2 = jnp.where(cand == m1, SMALL, cand)
        m2 = jnp.max(c2, axis=1, keepdims=True)
        c3 = jnp.where(c2 == m2, SMALL, c2)
        m3 = jnp.max(c3, axis=1, keepdims=True)

        def decode(key):
            dotv = (key + 32767) >> 15
            col = (dotv << 15) - key
            lab = (col * 8389) >> 24
            return jnp.where(dotv > -W, lab, 0)

        l1, l2, l3 = decode(m1), decode(m2), decode(m3)
        out_ref[...] = jnp.where(
            (l1 == l2) | (l1 == l3), l1,
            jnp.where(l2 == l3, l2, jnp.minimum(l1, jnp.minimum(l2, l3))),
        )


@jax.jit
def _knn(test_in, train_in, colvec):
    out = pl.pallas_call(
        _body,
        grid=(N_TILES, N_TEST // BT),
        in_specs=[
            pl.BlockSpec((BT, W), lambda t, i: (i, 0)),
            pl.BlockSpec((BC, W), lambda t, i: (t, 0)),
            pl.BlockSpec((1, BC), lambda t, i: (0, t)),
        ],
        out_specs=pl.BlockSpec((BT, 1), lambda t, i: (i, 0)),
        out_shape=jax.ShapeDtypeStruct((N_TEST, 1), jnp.int32),
        scratch_shapes=[pltpu.VMEM((N_TEST, LANES), jnp.int32)] * 3,
        compiler_params=pltpu.CompilerParams(
            dimension_semantics=("arbitrary", "arbitrary"),
        ),
    )(test_in, train_in, colvec)
    return out.reshape(N_TEST)


def kernel(training_set, test_set):
    test_in = (1 - 2 * test_set).astype(jnp.int8)
    train_in = (1 - 2 * training_set).astype(jnp.int8)
    train_in = jnp.pad(train_in, ((0, N_PAD - N_TRAIN), (0, 0)))
    j = jnp.arange(N_PAD, dtype=jnp.int32)
    colvec = jnp.where(j < N_TRAIN, j, 2 ** 25 + j).reshape(1, N_PAD)
    return _knn(test_in, train_in, colvec)


# speedup vs baseline: 1.2797x; 1.2797x over previous
import functools

import jax
import jax.numpy as jnp
from jax.experimental import pallas as pl
from jax.experimental.pallas import tpu as pltpu

N_TRAIN = 20000
W = 256
N_TEST = 1024
BC = 2048
N_PAD = 20480
N_TILES = N_PAD // BC
BT = 512
RB = 128
LANES = 128
CHUNKS = BC // LANES
CLASS_SIZE = 2000
NUM_CLASSES = 10
MAX_DISTANCE = 256
SCALE = 16384.0
BIG = 3.0e7


def _body(test_ref, train_ref, colv_ref, out_ref, t1, t2, t3):
    t = pl.program_id(0)
    base = pl.program_id(1) * BT
    dot = jax.lax.dot_general(
        test_ref[...], train_ref[...],
        (((1,), (1,)), ((), ())),
        preferred_element_type=jnp.float32,
    )

    @pl.when(t == 0)
    def _():
        sl = pl.ds(base, BT)
        t1[sl, :] = jnp.full((BT, LANES), BIG, jnp.float32)
        t2[sl, :] = jnp.full((BT, LANES), BIG, jnp.float32)
        t3[sl, :] = jnp.full((BT, LANES), BIG, jnp.float32)

    for rb in range(BT // RB):
        sl = pl.ds(base + rb * RB, RB)
        b1, b2, b3 = t1[sl, :], t2[sl, :], t3[sl, :]
        for c in range(CHUNKS):
            x = (dot[rb * RB:(rb + 1) * RB, c * LANES:(c + 1) * LANES]
                 + colv_ref[:, c * LANES:(c + 1) * LANES])
            lo = jnp.minimum(b1, x)
            hi = jnp.maximum(b1, x)
            b1 = lo
            lo = jnp.minimum(b2, hi)
            hi = jnp.maximum(b2, hi)
            b2 = lo
            b3 = jnp.minimum(b3, hi)
        t1[sl, :] = b1
        t2[sl, :] = b2
        t3[sl, :] = b3

    @pl.when(t == N_TILES - 1)
    def _():
        sl = pl.ds(base, BT)
        cand = jnp.concatenate([t1[sl, :], t2[sl, :], t3[sl, :]], axis=1)
        m1 = jnp.min(cand, axis=1, keepdims=True)
        c2 = jnp.where(cand == m1, BIG, cand)
        m2 = jnp.min(c2, axis=1, keepdims=True)
        c3 = jnp.where(c2 == m2, BIG, c2)
        m3 = jnp.min(c3, axis=1, keepdims=True)

        def decode(key_f):
            ki = key_f.astype(jnp.int32)
            dist = ki >> 15
            idx = ki & 32767
            lab = (idx * 8389) >> 24
            return jnp.where(dist < MAX_DISTANCE, lab, 0)

        l1, l2, l3 = decode(m1), decode(m2), decode(m3)
        out_ref[...] = jnp.where(
            (l1 == l2) | (l1 == l3), l1,
            jnp.where(l2 == l3, l2, jnp.minimum(l1, jnp.minimum(l2, l3))),
        )


@jax.jit
def _knn(test_in, train_in, colvec):
    out = pl.pallas_call(
        _body,
        grid=(N_TILES, N_TEST // BT),
        in_specs=[
            pl.BlockSpec((BT, W), lambda t, i: (i, 0)),
            pl.BlockSpec((BC, W), lambda t, i: (t, 0)),
            pl.BlockSpec((1, BC), lambda t, i: (0, t)),
        ],
        out_specs=pl.BlockSpec((BT, 1), lambda t, i: (i, 0)),
        out_shape=jax.ShapeDtypeStruct((N_TEST, 1), jnp.int32),
        scratch_shapes=[pltpu.VMEM((N_TEST, LANES), jnp.float32)] * 3,
        compiler_params=pltpu.CompilerParams(
            dimension_semantics=("arbitrary", "arbitrary"),
        ),
    )(test_in, train_in, colvec)
    return out.reshape(N_TEST)


def kernel(training_set, test_set):
    test_in = ((2 * test_set - 1) * 16384).astype(jnp.bfloat16)
    train_in = (1 - 2 * training_set).astype(jnp.bfloat16)
    train_in = jnp.pad(train_in, ((0, N_PAD - N_TRAIN), (0, 0)))
    j = jnp.arange(N_PAD, dtype=jnp.float32)
    colvec = jnp.where(j < N_TRAIN, SCALE * W + j, 2.5e7 + j).reshape(1, N_PAD)
    return _knn(test_in, train_in, colvec)
